# Initial kernel scaffold; baseline (speedup 1.0000x reference)
import jax
import jax.numpy as jnp
from jax.experimental import pallas as pl


def kernel(cut_coordinates, cut_reflatent_idx, cut_local_gene_ix, cut_local_cell_ix, cut_local_cellxgene_ix, cells_oi, n_cells, logit_weight, baseline, reflatent):
    raise NotImplementedError("write your pallas kernel here")


# speedup vs baseline: 13.3739x; 13.3739x over previous
import functools
import math

import jax
import jax.numpy as jnp
from jax import lax
from jax.experimental import pallas as pl
from jax.experimental.pallas import tpu as pltpu
from jax.experimental.pallas import tpu_sc as plsc

_N_TOTAL_CELLS = 10000.0

_NW = 32
_CHUNK = 128
_LANES = 16


def _table_kernel(refl_ref, wt_ref, base_ref, out_ref, kl_ref):
    mix = jnp.dot(refl_ref[...], wt_ref[...], preferred_element_type=jnp.float32)
    gb, k = base_ref.shape
    logits = mix.reshape(mix.shape[0], gb, k) + base_ref[...][None, :, :]
    m = jnp.max(logits, axis=-1, keepdims=True)
    ex = jnp.exp(logits - m)
    s = jnp.sum(ex, axis=-1, keepdims=True)
    out_ref[...] = logits - m - jnp.log(s)

    @pl.when(pl.program_id(0) == 0)
    def _():
        kl_ref[0, 0] = 0.0

    kl_ref[0, 0] += jnp.sum(wt_ref[...] * wt_ref[...])


def _build_table(reflatent, w_t, baseline, gene_block):
    r, l = reflatent.shape
    g, k = baseline.shape
    grid = g // gene_block
    return pl.pallas_call(
        _table_kernel,
        grid=(grid,),
        in_specs=[
            pl.BlockSpec((r, l), lambda i: (0, 0)),
            pl.BlockSpec((l, gene_block * k), lambda i: (0, i)),
            pl.BlockSpec((gene_block, k), lambda i: (i, 0)),
        ],
        out_specs=[
            pl.BlockSpec((r, gene_block, k), lambda i: (0, i, 0)),
            pl.BlockSpec(memory_space=pltpu.SMEM),
        ],
        out_shape=[
            jax.ShapeDtypeStruct((r, g, k), jnp.float32),
            jax.ShapeDtypeStruct((1, 1), jnp.float32),
        ],
    )(reflatent, w_t, baseline)


def _index_kernel(nbins, rowscale, r_ref, g_ref, c_ref, out_ref):
    b = (c_ref[...] * float(nbins)).astype(jnp.int32)
    b = jnp.clip(b, 0, nbins - 1)
    out_ref[...] = r_ref[...] * rowscale + g_ref[...] * nbins + b


def _build_indices(r2d, g2d, c2d, nbins, rowscale):
    return pl.pallas_call(
        functools.partial(_index_kernel, nbins, rowscale),
        out_shape=jax.ShapeDtypeStruct(r2d.shape, jnp.int32),
    )(r2d, g2d, c2d)


def _gather_sum_body(n_valid, idx_hbm, table_hbm, out_hbm, idx_v, val_v,
                     acc_v, sem):
    wid = lax.axis_index("s") * 2 + lax.axis_index("c")
    per = idx_v.shape[0]
    base = wid * per
    pltpu.sync_copy(idx_hbm.at[pl.ds(base, per)], idx_v)

    nchunks = per // _CHUNK

    def fire(i, carry):
        off = i * _CHUNK
        pltpu.async_copy(table_hbm.at[idx_v.at[pl.ds(off, _CHUNK)]],
                         val_v.at[pl.ds(off, _CHUNK)], sem)
        return carry

    lax.fori_loop(0, nchunks, fire, 0)

    def drain(i, carry):
        off = i * _CHUNK
        pltpu.make_async_copy(table_hbm.at[idx_v.at[pl.ds(off, _CHUNK)]],
                              val_v.at[pl.ds(off, _CHUNK)], sem).wait()
        return carry

    lax.fori_loop(0, nchunks, drain, 0)

    nvec = jnp.clip(n_valid - base, 0, per) // _LANES

    def body(i, acc):
        return acc + val_v[pl.ds(i * _LANES, _LANES)]

    acc = lax.fori_loop(0, nvec, body, jnp.zeros((_LANES,), jnp.float32))
    acc_v[...] = acc
    pltpu.sync_copy(acc_v, out_hbm.at[wid])


def _gather_sum(idx_flat, table_flat, n_valid):
    per = idx_flat.shape[0] // _NW
    mesh = plsc.VectorSubcoreMesh(core_axis_name="c", subcore_axis_name="s")
    kfn = functools.partial(
        pl.kernel,
        mesh=mesh,
        out_type=jax.ShapeDtypeStruct((_NW, _LANES), jnp.float32),
        scratch_types=[
            pltpu.VMEM((per,), jnp.int32),
            pltpu.VMEM((per,), jnp.float32),
            pltpu.VMEM((_LANES,), jnp.float32),
            pltpu.SemaphoreType.DMA,
        ],
    )(functools.partial(_gather_sum_body, n_valid))
    return kfn(idx_flat, table_flat)


def kernel(cut_coordinates, cut_reflatent_idx, cut_local_gene_ix,
           cut_local_cell_ix, cut_local_cellxgene_ix, cells_oi, n_cells,
           logit_weight, baseline, reflatent):
    g, l, k = logit_weight.shape
    r = reflatent.shape[0]
    n_cuts = cut_coordinates.shape[0]

    w_t = logit_weight.transpose(1, 0, 2).reshape(l, g * k)
    table, kl_sumsq = _build_table(reflatent, w_t, baseline, gene_block=40)

    unit = _NW * _CHUNK
    n_pad = (n_cuts + unit - 1) // unit * unit
    pad = n_pad - n_cuts
    rp = jnp.pad(cut_reflatent_idx.astype(jnp.int32), (0, pad))
    gp = jnp.pad(cut_local_gene_ix.astype(jnp.int32), (0, pad))
    cp = jnp.pad(cut_coordinates, (0, pad))
    rows = n_pad // _CHUNK
    idx2d = _build_indices(rp.reshape(rows, _CHUNK), gp.reshape(rows, _CHUNK),
                           cp.reshape(rows, _CHUNK), k, g * k)

    partials = _gather_sum(idx2d.reshape(n_pad), table.reshape(r * g * k),
                           n_cuts)

    logp_sum = jnp.sum(partials)
    likelihood = (logp_sum + jnp.float32(n_cuts * math.log(k)))
    likelihood = likelihood * jnp.float32(_N_TOTAL_CELLS) / n_cells
    kl = (-0.5 * kl_sumsq[0, 0]
          - jnp.float32(0.5 * math.log(2.0 * math.pi) * g * l * k))
    elbo = -likelihood - kl
    return elbo / jnp.float32(_N_TOTAL_CELLS)
